# Initial kernel scaffold; baseline (speedup 1.0000x reference)
import functools

import jax
import jax.numpy as jnp
from jax import lax
from jax.experimental import pallas as pl
from jax.experimental.pallas import tpu as pltpu
from jax.experimental.pallas import tpu_sc as plsc

NUM_ROWS = 1000000
D = 32

NC, NS = 2, 16
NW = NC * NS
G = 128
B_TOTAL = 16384 * 26
B_PER_W = B_TOTAL // NW
C = 1024
NCHUNK = B_PER_W // C
GPC = C // G


@functools.partial(
    pl.kernel,
    out_type=jax.ShapeDtypeStruct((B_TOTAL, D), jnp.float32),
    mesh=plsc.VectorSubcoreMesh(core_axis_name="c", subcore_axis_name="s"),
    scratch_types=[
        pltpu.VMEM((GPC, G), jnp.int32),
        pltpu.VMEM((C, D), jnp.float32),
        pltpu.SemaphoreType.DMA,
    ],
)
def _gather_kernel(idx_hbm, table_hbm, out_hbm, idx_v, rows_v, sem):
    wid = lax.axis_index("s") * NC + lax.axis_index("c")
    row_base = wid * B_PER_W

    @pl.loop(0, NCHUNK)
    def _chunk(ci):
        off = row_base + ci * C
        pltpu.sync_copy(idx_hbm.at[pl.ds(off // G, GPC)], idx_v)
        copies = []
        for j in range(GPC):
            copies.append(
                pltpu.async_copy(
                    table_hbm.at[idx_v.at[j]],
                    rows_v.at[pl.ds(j * G, G)],
                    sem,
                )
            )
        for cp in copies:
            cp.wait()
        pltpu.sync_copy(rows_v, out_hbm.at[pl.ds(off, C)])


def kernel(x, table):
    idx = x.reshape(B_TOTAL // G, G).astype(jnp.int32)
    out = _gather_kernel(idx, table)
    return out.reshape(x.shape + (D,))


# speedup vs baseline: 1.5468x; 1.5468x over previous
import functools

import jax
import jax.numpy as jnp
from jax import lax
from jax.experimental import pallas as pl
from jax.experimental.pallas import tpu as pltpu
from jax.experimental.pallas import tpu_sc as plsc

NUM_ROWS = 1000000
D = 32

NC, NS = 2, 16
NW = NC * NS
G = 128
B_TOTAL = 16384 * 26
B_PER_W = B_TOTAL // NW
C = 1024
NCHUNK = B_PER_W // C
GPC = C // G


@functools.partial(
    pl.kernel,
    out_type=jax.ShapeDtypeStruct((B_TOTAL, D), jnp.float32),
    mesh=plsc.VectorSubcoreMesh(core_axis_name="c", subcore_axis_name="s"),
    scratch_types=[
        pltpu.VMEM((GPC, G), jnp.int32),
        pltpu.VMEM((C, D), jnp.float32),
        pltpu.SemaphoreType.DMA,
    ],
    compiler_params=pltpu.CompilerParams(use_tc_tiling_on_sc=False),
)
def _gather_kernel(idx_hbm, table_hbm, out_hbm, idx_v, rows_v, sem):
    wid = lax.axis_index("s") * NC + lax.axis_index("c")
    row_base = wid * B_PER_W

    @pl.loop(0, NCHUNK)
    def _chunk(ci):
        off = pl.multiple_of(row_base + ci * C, C)
        pltpu.sync_copy(idx_hbm.at[pl.ds(pl.multiple_of(off // G, GPC), GPC)], idx_v)
        copies = []
        for j in range(GPC):
            copies.append(
                pltpu.async_copy(
                    table_hbm.at[idx_v.at[j]],
                    rows_v.at[pl.ds(j * G, G)],
                    sem,
                )
            )
        for cp in copies:
            cp.wait()
        pltpu.sync_copy(rows_v, out_hbm.at[pl.ds(off, C)])


def kernel(x, table):
    idx = x.reshape(B_TOTAL // G, G).astype(jnp.int32)
    out = _gather_kernel(idx, table)
    return out.reshape(x.shape + (D,))


# speedup vs baseline: 1.5677x; 1.0135x over previous
import functools

import jax
import jax.numpy as jnp
from jax import lax
from jax.experimental import pallas as pl
from jax.experimental.pallas import tpu as pltpu
from jax.experimental.pallas import tpu_sc as plsc

NUM_ROWS = 1000000
D = 32

NC, NS = 2, 16
NW = NC * NS
G = 128
B_TOTAL = 16384 * 26
B_PER_W = B_TOTAL // NW
C = 1664
NCHUNK = B_PER_W // C
GPC = C // G
IDX_ROWS = B_PER_W // G


@functools.partial(
    pl.kernel,
    out_type=jax.ShapeDtypeStruct((B_TOTAL, D), jnp.float32),
    mesh=plsc.VectorSubcoreMesh(core_axis_name="c", subcore_axis_name="s"),
    scratch_types=[
        pltpu.VMEM((IDX_ROWS, G), jnp.int32),
        pltpu.VMEM((2, C, D), jnp.float32),
        pltpu.SemaphoreType.DMA,
        pltpu.SemaphoreType.DMA,
        pltpu.SemaphoreType.DMA,
    ],
    compiler_params=pltpu.CompilerParams(use_tc_tiling_on_sc=False),
)
def _gather_kernel(idx_hbm, table_hbm, out_hbm, idx_v, rows_v, gsem,
                   osem0, osem1):
    wid = lax.axis_index("s") * NC + lax.axis_index("c")
    row_base = wid * B_PER_W

    pltpu.sync_copy(
        idx_hbm.at[pl.ds(pl.multiple_of(row_base // G, IDX_ROWS), IDX_ROWS)],
        idx_v,
    )

    @pl.loop(0, NCHUNK)
    def _chunk(ci):
        b = lax.rem(ci, 2)
        off = pl.multiple_of(row_base + ci * C, C)

        @pl.when(ci >= 2)
        def _():
            @pl.when(b == 0)
            def _():
                pltpu.make_async_copy(
                    out_hbm.at[pl.ds(0, C)], rows_v.at[0], osem0
                ).wait()

            @pl.when(b == 1)
            def _():
                pltpu.make_async_copy(
                    out_hbm.at[pl.ds(0, C)], rows_v.at[1], osem1
                ).wait()

        copies = []
        for j in range(GPC):
            copies.append(
                pltpu.async_copy(
                    table_hbm.at[idx_v.at[ci * GPC + j]],
                    rows_v.at[b, pl.ds(j * G, G)],
                    gsem,
                )
            )
        for cp in copies:
            cp.wait()

        @pl.when(b == 0)
        def _():
            pltpu.async_copy(rows_v.at[0], out_hbm.at[pl.ds(off, C)], osem0)

        @pl.when(b == 1)
        def _():
            pltpu.async_copy(rows_v.at[1], out_hbm.at[pl.ds(off, C)], osem1)

    pltpu.make_async_copy(out_hbm.at[pl.ds(0, C)], rows_v.at[0], osem0).wait()
    pltpu.make_async_copy(out_hbm.at[pl.ds(0, C)], rows_v.at[1], osem1).wait()


def kernel(x, table):
    idx = x.reshape(B_TOTAL // G, G).astype(jnp.int32)
    out = _gather_kernel(idx, table)
    return out.reshape(x.shape + (D,))
